# baseline (speedup 1.0000x reference)
import functools

import jax
import jax.numpy as jnp
from jax import lax
from jax.experimental import pallas as pl
from jax.experimental.pallas import tpu as pltpu
from jax.experimental.pallas import tpu_sc as plsc

N_NODES = 10000
N_EDGES = 160000
D_FEAT = 256
HALF = 128
N_CLASSES = 10
N_GRAPHS = 8
EPS = 1e-15

ROW_BLK = 200
N_ROW_BLKS = N_NODES // ROW_BLK
EM_BLK = N_EDGES // N_ROW_BLKS

CHUNK = 128
N_SUB = 16
SUB_ROWS = 640
ACC_ROWS = N_SUB * SUB_ROWS
IDX_BLK = 16
PAD_EDGES = N_SUB * IDX_BLK * CHUNK
DST_PAD = N_NODES


def _segsum_body(meta, ga_hbm, gb_hbm, src_hbm, dst_hbm, em_hbm,
                 outa_hbm, outb_hbm, acc, srcv, dstv, emv, rows, sem):
    n_nodes, n_edges, half = meta
    n_chunks = n_edges // CHUNK
    per_tile = n_chunks // N_SUB
    nf = half // 16

    c = lax.axis_index("c")
    s = lax.axis_index("s")

    zero = jnp.zeros((16,), jnp.float32)

    def zrow(r, carry):
        for f in range(nf):
            rows[r, pl.ds(16 * f, 16)] = zero
        return carry

    lax.fori_loop(0, CHUNK, zrow, 0)
    my_rows = pl.multiple_of(s * SUB_ROWS, SUB_ROWS)
    for j in range(SUB_ROWS // CHUNK):
        pltpu.sync_copy(rows.at[pl.ds(0, CHUNK)],
                        acc.at[pl.ds(my_rows + j * CHUNK, CHUNK)])
    plsc.subcore_barrier()

    def run(g_hbm, out_hbm):
        def chunk_body(k):
            base = pl.multiple_of(k * CHUNK, CHUNK)
            pltpu.sync_copy(src_hbm.at[pl.ds(base, CHUNK)], srcv)
            pltpu.sync_copy(dst_hbm.at[pl.ds(base, CHUNK)], dstv)
            pltpu.sync_copy(em_hbm.at[pl.ds(base, CHUNK)], emv)
            pltpu.async_copy(g_hbm.at[srcv], rows, sem).wait()

            def scale_group(g, carry):
                emg = emv[pl.ds(pl.multiple_of(g * 16, 16), 16)]
                for j in range(16):
                    scal = jnp.full((16,), emg[j])
                    e = g * 16 + j
                    for f in range(nf):
                        rows[e, pl.ds(16 * f, 16)] = rows[e, pl.ds(16 * f, 16)] * scal
                return carry

            lax.fori_loop(0, CHUNK // 16, scale_group, 0)
            pltpu.sync_copy(rows, acc.at[dstv], add=True)

        def loop(i, carry):
            chunk_body(i * N_SUB + s)
            return carry

        lax.fori_loop(0, per_tile, loop, 0)
        plsc.subcore_barrier()
        out_count_full = SUB_ROWS
        out_count_last = n_nodes - (N_SUB - 1) * SUB_ROWS

        @pl.when(s < N_SUB - 1)
        def _():
            pltpu.sync_copy(acc.at[pl.ds(my_rows, out_count_full)],
                            out_hbm.at[pl.ds(my_rows, out_count_full)])

        @pl.when(s == N_SUB - 1)
        def _():
            base_last = (N_SUB - 1) * SUB_ROWS
            pltpu.sync_copy(acc.at[pl.ds(base_last, out_count_last)],
                            out_hbm.at[pl.ds(base_last, out_count_last)])

    @pl.when(c == 0)
    def _():
        run(ga_hbm, outa_hbm)

    @pl.when(c == 1)
    def _():
        run(gb_hbm, outb_hbm)


@functools.lru_cache(maxsize=None)
def _build_segsum(n_nodes, n_edges, half):
    mesh = plsc.VectorSubcoreMesh(core_axis_name="c", subcore_axis_name="s")
    return pl.kernel(
        functools.partial(_segsum_body, (n_nodes, n_edges, half)),
        out_type=(jax.ShapeDtypeStruct((n_nodes, half), jnp.float32),
                  jax.ShapeDtypeStruct((n_nodes, half), jnp.float32)),
        mesh=mesh,
        scratch_types=[
            pltpu.VMEM_SHARED((ACC_ROWS, half), jnp.float32),
            pltpu.VMEM((CHUNK,), jnp.int32),
            pltpu.VMEM((CHUNK,), jnp.int32),
            pltpu.VMEM((CHUNK,), jnp.float32),
            pltpu.VMEM((CHUNK, half), jnp.float32),
            pltpu.SemaphoreType.DMA,
        ],
        name="segsum_sc",
    )


def _segsum_sc(ga, gb, src_p, dst_p, em_p):
    return _build_segsum(ga.shape[0], src_p.shape[0], ga.shape[1])(
        ga, gb, src_p, dst_p, em_p)


def _pad_edges(src, dst, em_sig):
    n_edges = src.shape[0]
    n_pad = (-n_edges) % PAD_EDGES
    src_p = jnp.concatenate([src, jnp.zeros((n_pad,), jnp.int32)])
    dst_p = jnp.concatenate([dst, jnp.full((n_pad,), DST_PAD, jnp.int32)])
    em_p = jnp.concatenate([em_sig, jnp.zeros((n_pad,), jnp.float32)])
    return src_p, dst_p, em_p


def _stage1_body(x_ref, nfm_ref, em_ref, w1a_ref, w1b_ref, ga_ref, gb_ref, ems_ref):
    sfm = jax.nn.sigmoid(nfm_ref[...])
    h = x_ref[...] * sfm
    ga_ref[...] = jnp.dot(h, w1a_ref[...], preferred_element_type=jnp.float32)
    gb_ref[...] = jnp.dot(h, w1b_ref[...], preferred_element_type=jnp.float32)
    ems_ref[...] = jax.nn.sigmoid(em_ref[...])


def _stage1(x, nfm2, em3, w1a, w1b):
    return pl.pallas_call(
        _stage1_body,
        grid=(N_ROW_BLKS,),
        in_specs=[
            pl.BlockSpec((ROW_BLK, D_FEAT), lambda i: (i, 0)),
            pl.BlockSpec((1, D_FEAT), lambda i: (0, 0)),
            pl.BlockSpec((1, 1, EM_BLK), lambda i: (i, 0, 0)),
            pl.BlockSpec((D_FEAT, HALF), lambda i: (0, 0)),
            pl.BlockSpec((D_FEAT, HALF), lambda i: (0, 0)),
        ],
        out_specs=[
            pl.BlockSpec((ROW_BLK, HALF), lambda i: (i, 0)),
            pl.BlockSpec((ROW_BLK, HALF), lambda i: (i, 0)),
            pl.BlockSpec((1, 1, EM_BLK), lambda i: (i, 0, 0)),
        ],
        out_shape=[
            jax.ShapeDtypeStruct((N_NODES, HALF), jnp.float32),
            jax.ShapeDtypeStruct((N_NODES, HALF), jnp.float32),
            jax.ShapeDtypeStruct((N_ROW_BLKS, 1, EM_BLK), jnp.float32),
        ],
    )(x, nfm2, em3, w1a, w1b)


def _stage2_body(aa_ref, ab_ref, w2_ref, ga_ref, gb_ref):
    h1a = jnp.maximum(aa_ref[...], 0.0)
    h1b = jnp.maximum(ab_ref[...], 0.0)
    w2 = w2_ref[...]
    ga_ref[...] = (jnp.dot(h1a, w2[:HALF, :HALF], preferred_element_type=jnp.float32)
                   + jnp.dot(h1b, w2[HALF:, :HALF], preferred_element_type=jnp.float32))
    gb_ref[...] = (jnp.dot(h1a, w2[:HALF, HALF:], preferred_element_type=jnp.float32)
                   + jnp.dot(h1b, w2[HALF:, HALF:], preferred_element_type=jnp.float32))


def _stage2(aa, ab, w2):
    return pl.pallas_call(
        _stage2_body,
        grid=(N_ROW_BLKS,),
        in_specs=[
            pl.BlockSpec((ROW_BLK, HALF), lambda i: (i, 0)),
            pl.BlockSpec((ROW_BLK, HALF), lambda i: (i, 0)),
            pl.BlockSpec((D_FEAT, D_FEAT), lambda i: (0, 0)),
        ],
        out_specs=[
            pl.BlockSpec((ROW_BLK, HALF), lambda i: (i, 0)),
            pl.BlockSpec((ROW_BLK, HALF), lambda i: (i, 0)),
        ],
        out_shape=[
            jax.ShapeDtypeStruct((N_NODES, HALF), jnp.float32),
            jax.ShapeDtypeStruct((N_NODES, HALF), jnp.float32),
        ],
    )(aa, ab, w2)


def _stage3_body(aa_ref, ab_ref, wout_ref, bi_ref, em_ref, pool_ref, stats_ref):
    i = pl.program_id(0)

    @pl.when(i == 0)
    def _():
        pool_ref[...] = jnp.zeros_like(pool_ref)
        stats_ref[...] = jnp.zeros_like(stats_ref)

    h2a = jnp.maximum(aa_ref[...], 0.0)
    h2b = jnp.maximum(ab_ref[...], 0.0)
    w = wout_ref[...]
    z = (jnp.dot(h2a, w[:HALF], preferred_element_type=jnp.float32)
         + jnp.dot(h2b, w[HALF:], preferred_element_type=jnp.float32))
    bi = bi_ref[0, 0, :]
    graphs = lax.broadcasted_iota(jnp.int32, (ROW_BLK, N_GRAPHS), 1)
    onehot = (bi[:, None] == graphs).astype(jnp.float32)
    zc = jnp.concatenate(
        [z, jnp.ones((ROW_BLK, 1), jnp.float32), jnp.zeros((ROW_BLK, 5), jnp.float32)],
        axis=1)
    pool_ref[...] += lax.dot_general(onehot, zc, (((0,), (0,)), ((), ())),
                                     preferred_element_type=jnp.float32)
    em = em_ref[...]
    s_em = jnp.sum(em)
    ent = -em * jnp.log(em + EPS) - (1.0 - em) * jnp.log(1.0 - em + EPS)
    s_ent = jnp.sum(ent)
    lane = lax.broadcasted_iota(jnp.int32, (1, 128), 1)
    stats_ref[...] += (jnp.where(lane == 0, s_em, 0.0)
                       + jnp.where(lane == 1, s_ent, 0.0))


def _stage3(aa, ab, wout, bi3, em3):
    return pl.pallas_call(
        _stage3_body,
        grid=(N_ROW_BLKS,),
        in_specs=[
            pl.BlockSpec((ROW_BLK, HALF), lambda i: (i, 0)),
            pl.BlockSpec((ROW_BLK, HALF), lambda i: (i, 0)),
            pl.BlockSpec((D_FEAT, N_CLASSES), lambda i: (0, 0)),
            pl.BlockSpec((1, 1, ROW_BLK), lambda i: (i, 0, 0)),
            pl.BlockSpec((1, 1, EM_BLK), lambda i: (i, 0, 0)),
        ],
        out_specs=[
            pl.BlockSpec((N_GRAPHS, 16), lambda i: (0, 0)),
            pl.BlockSpec((1, 128), lambda i: (0, 0)),
        ],
        out_shape=[
            jax.ShapeDtypeStruct((N_GRAPHS, 16), jnp.float32),
            jax.ShapeDtypeStruct((1, 128), jnp.float32),
        ],
    )(aa, ab, wout, bi3, em3)


def _stage4_body(pool_ref, stats_ref, nfm_ref, label_ref, out_ref):
    pool = pool_ref[...]
    counts = jnp.maximum(pool[:, 10:11], 1.0)
    logits = pool[:, :N_CLASSES] / counts
    mx = jnp.max(logits, axis=1, keepdims=True)
    ex = jnp.exp(logits - mx)
    lse = jnp.log(jnp.sum(ex, axis=1, keepdims=True)) + mx
    lbl = label_ref[0, 0]
    cls = lax.broadcasted_iota(jnp.int32, (N_GRAPHS, N_CLASSES), 1)
    sel = jnp.sum(jnp.where(cls == lbl, logits, 0.0), axis=1, keepdims=True)
    loss_pred = jnp.sum(lse - sel)
    s_em = stats_ref[0, 0]
    s_ent = stats_ref[0, 1]
    fm = jax.nn.sigmoid(nfm_ref[...])
    reg = 0.1 * s_em + s_ent / N_EDGES + jnp.mean(fm)
    loss = loss_pred + N_GRAPHS * reg
    out_ref[...] = jnp.broadcast_to(loss, (1, 1))


def _stage4(pool, stats, nfm2, label):
    return pl.pallas_call(
        _stage4_body,
        out_shape=jax.ShapeDtypeStruct((1, 1), jnp.float32),
    )(pool, stats, nfm2, label)


def kernel(x, edge_index, batch_index, expl_label, node_feat_mask, edge_mask,
           W1, W2, W_out):
    src = edge_index[0]
    dst = edge_index[1]
    nfm2 = node_feat_mask.reshape(1, D_FEAT)
    em3 = edge_mask.reshape(N_ROW_BLKS, 1, EM_BLK)
    ga, gb, ems3 = _stage1(x, nfm2, em3, W1[:, :HALF], W1[:, HALF:])
    em_sig = ems3.reshape(N_EDGES)
    src2, dst2, em2 = _pad_edges(src, dst, em_sig)
    aa, ab = _segsum_sc(ga, gb, src2, dst2, em2)
    g2a, g2b = _stage2(aa, ab, W2)
    a2a, a2b = _segsum_sc(g2a, g2b, src2, dst2, em2)
    bi3 = batch_index.reshape(N_ROW_BLKS, 1, ROW_BLK)
    pool, stats = _stage3(a2a, a2b, W_out, bi3, ems3)
    label = jnp.asarray(expl_label, jnp.int32).reshape(1, 1)
    out = _stage4(pool, stats, nfm2, label)
    return out.reshape(())


# speedup vs baseline: 1.6291x; 1.6291x over previous
import functools

import jax
import jax.numpy as jnp
from jax import lax
from jax.experimental import pallas as pl
from jax.experimental.pallas import tpu as pltpu
from jax.experimental.pallas import tpu_sc as plsc

N_NODES = 10000
N_EDGES = 160000
D_FEAT = 256
HALF = 128
N_CLASSES = 10
N_GRAPHS = 8
EPS = 1e-15

ROW_BLK = 200
N_ROW_BLKS = N_NODES // ROW_BLK
EM_BLK = N_EDGES // N_ROW_BLKS

CHUNK = 128
N_SUB = 16
SUB_ROWS = 640
ACC_ROWS = N_SUB * SUB_ROWS
IDX_BLK = 16
PAD_EDGES = N_SUB * IDX_BLK * CHUNK
DST_PAD = N_NODES


def _segsum_body(meta, ga_hbm, gb_hbm, idx_hbm,
                 outa_hbm, outb_hbm, acc, idxv, rows, sem):
    n_nodes, n_edges, half = meta
    n_chunks = n_edges // CHUNK
    base_chunks = n_chunks // N_SUB
    rem_chunks = n_chunks % N_SUB
    nf = half // 16

    c = lax.axis_index("c")
    s = lax.axis_index("s")

    zero = jnp.zeros((16,), jnp.float32)

    def zrow(r, carry):
        for f in range(nf):
            rows[r, pl.ds(16 * f, 16)] = zero
        return carry

    lax.fori_loop(0, CHUNK, zrow, 0)
    my_rows = pl.multiple_of(s * SUB_ROWS, SUB_ROWS)
    for j in range(SUB_ROWS // CHUNK):
        pltpu.sync_copy(rows.at[pl.ds(0, CHUNK)],
                        acc.at[pl.ds(my_rows + j * CHUNK, CHUNK)])
    plsc.subcore_barrier()

    n_mine = base_chunks + (s < rem_chunks).astype(jnp.int32)

    def run(g_hbm, out_hbm):
        def chunk_body(k):
            pltpu.sync_copy(idx_hbm.at[k], idxv)
            pltpu.async_copy(g_hbm.at[idxv.at[0]], rows, sem).wait()

            def scale_group(g, carry):
                emg = idxv[2, pl.ds(pl.multiple_of(g * 16, 16), 16)]
                for j in range(16):
                    scal = jnp.full(
                        (16,), lax.bitcast_convert_type(emg[j], jnp.float32))
                    e = g * 16 + j
                    for f in range(nf):
                        rows[e, pl.ds(16 * f, 16)] = rows[e, pl.ds(16 * f, 16)] * scal
                return carry

            lax.fori_loop(0, CHUNK // 16, scale_group, 0)
            pltpu.sync_copy(rows, acc.at[idxv.at[1]], add=True)

        def loop(i, carry):
            chunk_body(i * N_SUB + s)
            return carry

        lax.fori_loop(0, n_mine, loop, 0)
        plsc.subcore_barrier()
        out_count_full = SUB_ROWS
        out_count_last = n_nodes - (N_SUB - 1) * SUB_ROWS

        @pl.when(s < N_SUB - 1)
        def _():
            pltpu.sync_copy(acc.at[pl.ds(my_rows, out_count_full)],
                            out_hbm.at[pl.ds(my_rows, out_count_full)])

        @pl.when(s == N_SUB - 1)
        def _():
            base_last = (N_SUB - 1) * SUB_ROWS
            pltpu.sync_copy(acc.at[pl.ds(base_last, out_count_last)],
                            out_hbm.at[pl.ds(base_last, out_count_last)])

    @pl.when(c == 0)
    def _():
        run(ga_hbm, outa_hbm)

    @pl.when(c == 1)
    def _():
        run(gb_hbm, outb_hbm)


@functools.lru_cache(maxsize=None)
def _build_segsum(n_nodes, n_edges, half):
    mesh = plsc.VectorSubcoreMesh(core_axis_name="c", subcore_axis_name="s")
    return pl.kernel(
        functools.partial(_segsum_body, (n_nodes, n_edges, half)),
        out_type=(jax.ShapeDtypeStruct((n_nodes, half), jnp.float32),
                  jax.ShapeDtypeStruct((n_nodes, half), jnp.float32)),
        mesh=mesh,
        scratch_types=[
            pltpu.VMEM_SHARED((ACC_ROWS, half), jnp.float32),
            pltpu.VMEM((3, CHUNK), jnp.int32),
            pltpu.VMEM((CHUNK, half), jnp.float32),
            pltpu.SemaphoreType.DMA,
        ],
        name="segsum_sc",
    )


def _segsum_sc(ga, gb, idx3):
    return _build_segsum(ga.shape[0], idx3.shape[0] * CHUNK, ga.shape[1])(
        ga, gb, idx3)


def _pack_edges(src, dst, em_sig):
    n_chunks = src.shape[0] // CHUNK
    em_bits = lax.bitcast_convert_type(em_sig, jnp.int32)
    return jnp.stack([src.reshape(n_chunks, CHUNK), dst.reshape(n_chunks, CHUNK),
                      em_bits.reshape(n_chunks, CHUNK)], axis=1)


def _stage1_body(x_ref, nfm_ref, em_ref, w1a_ref, w1b_ref, ga_ref, gb_ref, ems_ref):
    sfm = jax.nn.sigmoid(nfm_ref[...])
    h = x_ref[...] * sfm
    ga_ref[...] = jnp.dot(h, w1a_ref[...], preferred_element_type=jnp.float32)
    gb_ref[...] = jnp.dot(h, w1b_ref[...], preferred_element_type=jnp.float32)
    ems_ref[...] = jax.nn.sigmoid(em_ref[...])


def _stage1(x, nfm2, em3, w1a, w1b):
    return pl.pallas_call(
        _stage1_body,
        grid=(N_ROW_BLKS,),
        in_specs=[
            pl.BlockSpec((ROW_BLK, D_FEAT), lambda i: (i, 0)),
            pl.BlockSpec((1, D_FEAT), lambda i: (0, 0)),
            pl.BlockSpec((1, 1, EM_BLK), lambda i: (i, 0, 0)),
            pl.BlockSpec((D_FEAT, HALF), lambda i: (0, 0)),
            pl.BlockSpec((D_FEAT, HALF), lambda i: (0, 0)),
        ],
        out_specs=[
            pl.BlockSpec((ROW_BLK, HALF), lambda i: (i, 0)),
            pl.BlockSpec((ROW_BLK, HALF), lambda i: (i, 0)),
            pl.BlockSpec((1, 1, EM_BLK), lambda i: (i, 0, 0)),
        ],
        out_shape=[
            jax.ShapeDtypeStruct((N_NODES, HALF), jnp.float32),
            jax.ShapeDtypeStruct((N_NODES, HALF), jnp.float32),
            jax.ShapeDtypeStruct((N_ROW_BLKS, 1, EM_BLK), jnp.float32),
        ],
    )(x, nfm2, em3, w1a, w1b)


def _stage2_body(aa_ref, ab_ref, w2_ref, ga_ref, gb_ref):
    h1a = jnp.maximum(aa_ref[...], 0.0)
    h1b = jnp.maximum(ab_ref[...], 0.0)
    w2 = w2_ref[...]
    ga_ref[...] = (jnp.dot(h1a, w2[:HALF, :HALF], preferred_element_type=jnp.float32)
                   + jnp.dot(h1b, w2[HALF:, :HALF], preferred_element_type=jnp.float32))
    gb_ref[...] = (jnp.dot(h1a, w2[:HALF, HALF:], preferred_element_type=jnp.float32)
                   + jnp.dot(h1b, w2[HALF:, HALF:], preferred_element_type=jnp.float32))


def _stage2(aa, ab, w2):
    return pl.pallas_call(
        _stage2_body,
        grid=(N_ROW_BLKS,),
        in_specs=[
            pl.BlockSpec((ROW_BLK, HALF), lambda i: (i, 0)),
            pl.BlockSpec((ROW_BLK, HALF), lambda i: (i, 0)),
            pl.BlockSpec((D_FEAT, D_FEAT), lambda i: (0, 0)),
        ],
        out_specs=[
            pl.BlockSpec((ROW_BLK, HALF), lambda i: (i, 0)),
            pl.BlockSpec((ROW_BLK, HALF), lambda i: (i, 0)),
        ],
        out_shape=[
            jax.ShapeDtypeStruct((N_NODES, HALF), jnp.float32),
            jax.ShapeDtypeStruct((N_NODES, HALF), jnp.float32),
        ],
    )(aa, ab, w2)


def _stage3_body(aa_ref, ab_ref, wout_ref, bi_ref, em_ref, pool_ref, stats_ref):
    i = pl.program_id(0)

    @pl.when(i == 0)
    def _():
        pool_ref[...] = jnp.zeros_like(pool_ref)
        stats_ref[...] = jnp.zeros_like(stats_ref)

    h2a = jnp.maximum(aa_ref[...], 0.0)
    h2b = jnp.maximum(ab_ref[...], 0.0)
    w = wout_ref[...]
    z = (jnp.dot(h2a, w[:HALF], preferred_element_type=jnp.float32)
         + jnp.dot(h2b, w[HALF:], preferred_element_type=jnp.float32))
    bi = bi_ref[0, 0, :]
    graphs = lax.broadcasted_iota(jnp.int32, (ROW_BLK, N_GRAPHS), 1)
    onehot = (bi[:, None] == graphs).astype(jnp.float32)
    zc = jnp.concatenate(
        [z, jnp.ones((ROW_BLK, 1), jnp.float32), jnp.zeros((ROW_BLK, 5), jnp.float32)],
        axis=1)
    pool_ref[...] += lax.dot_general(onehot, zc, (((0,), (0,)), ((), ())),
                                     preferred_element_type=jnp.float32)
    em = em_ref[...]
    s_em = jnp.sum(em)
    ent = -em * jnp.log(em + EPS) - (1.0 - em) * jnp.log(1.0 - em + EPS)
    s_ent = jnp.sum(ent)
    lane = lax.broadcasted_iota(jnp.int32, (1, 128), 1)
    stats_ref[...] += (jnp.where(lane == 0, s_em, 0.0)
                       + jnp.where(lane == 1, s_ent, 0.0))


def _stage3(aa, ab, wout, bi3, em3):
    return pl.pallas_call(
        _stage3_body,
        grid=(N_ROW_BLKS,),
        in_specs=[
            pl.BlockSpec((ROW_BLK, HALF), lambda i: (i, 0)),
            pl.BlockSpec((ROW_BLK, HALF), lambda i: (i, 0)),
            pl.BlockSpec((D_FEAT, N_CLASSES), lambda i: (0, 0)),
            pl.BlockSpec((1, 1, ROW_BLK), lambda i: (i, 0, 0)),
            pl.BlockSpec((1, 1, EM_BLK), lambda i: (i, 0, 0)),
        ],
        out_specs=[
            pl.BlockSpec((N_GRAPHS, 16), lambda i: (0, 0)),
            pl.BlockSpec((1, 128), lambda i: (0, 0)),
        ],
        out_shape=[
            jax.ShapeDtypeStruct((N_GRAPHS, 16), jnp.float32),
            jax.ShapeDtypeStruct((1, 128), jnp.float32),
        ],
    )(aa, ab, wout, bi3, em3)


def _stage4_body(pool_ref, stats_ref, nfm_ref, label_ref, out_ref):
    pool = pool_ref[...]
    counts = jnp.maximum(pool[:, 10:11], 1.0)
    logits = pool[:, :N_CLASSES] / counts
    mx = jnp.max(logits, axis=1, keepdims=True)
    ex = jnp.exp(logits - mx)
    lse = jnp.log(jnp.sum(ex, axis=1, keepdims=True)) + mx
    lbl = label_ref[0, 0]
    cls = lax.broadcasted_iota(jnp.int32, (N_GRAPHS, N_CLASSES), 1)
    sel = jnp.sum(jnp.where(cls == lbl, logits, 0.0), axis=1, keepdims=True)
    loss_pred = jnp.sum(lse - sel)
    s_em = stats_ref[0, 0]
    s_ent = stats_ref[0, 1]
    fm = jax.nn.sigmoid(nfm_ref[...])
    reg = 0.1 * s_em + s_ent / N_EDGES + jnp.mean(fm)
    loss = loss_pred + N_GRAPHS * reg
    out_ref[...] = jnp.broadcast_to(loss, (1, 1))


def _stage4(pool, stats, nfm2, label):
    return pl.pallas_call(
        _stage4_body,
        out_shape=jax.ShapeDtypeStruct((1, 1), jnp.float32),
    )(pool, stats, nfm2, label)


def kernel(x, edge_index, batch_index, expl_label, node_feat_mask, edge_mask,
           W1, W2, W_out):
    src = edge_index[0]
    dst = edge_index[1]
    nfm2 = node_feat_mask.reshape(1, D_FEAT)
    em3 = edge_mask.reshape(N_ROW_BLKS, 1, EM_BLK)
    ga, gb, ems3 = _stage1(x, nfm2, em3, W1[:, :HALF], W1[:, HALF:])
    em_sig = ems3.reshape(N_EDGES)
    idx3 = _pack_edges(src, dst, em_sig)
    aa, ab = _segsum_sc(ga, gb, idx3)
    g2a, g2b = _stage2(aa, ab, W2)
    a2a, a2b = _segsum_sc(g2a, g2b, idx3)
    bi3 = batch_index.reshape(N_ROW_BLKS, 1, ROW_BLK)
    pool, stats = _stage3(a2a, a2b, W_out, bi3, ems3)
    label = jnp.asarray(expl_label, jnp.int32).reshape(1, 1)
    out = _stage4(pool, stats, nfm2, label)
    return out.reshape(())


# speedup vs baseline: 1.8461x; 1.1332x over previous
import functools

import jax
import jax.numpy as jnp
from jax import lax
from jax.experimental import pallas as pl
from jax.experimental.pallas import tpu as pltpu
from jax.experimental.pallas import tpu_sc as plsc

N_NODES = 10000
N_EDGES = 160000
D_FEAT = 256
HALF = 128
N_CLASSES = 10
N_GRAPHS = 8
EPS = 1e-15

ROW_BLK = 200
N_ROW_BLKS = N_NODES // ROW_BLK
EM_BLK = N_EDGES // N_ROW_BLKS

CHUNK = 128
N_SUB = 16
SUB_ROWS = 640
ACC_ROWS = N_SUB * SUB_ROWS
IDX_BLK = 16
PAD_EDGES = N_SUB * IDX_BLK * CHUNK
DST_PAD = N_NODES


def _segsum_body(meta, ga_hbm, gb_hbm, idx_hbm,
                 outa_hbm, outb_hbm, acc, idxv0, idxv1, rows,
                 gsem, isem0, isem1):
    n_nodes, n_edges, half = meta
    n_chunks = n_edges // CHUNK
    base_chunks = n_chunks // N_SUB
    rem_chunks = n_chunks % N_SUB
    nf = half // 16

    c = lax.axis_index("c")
    s = lax.axis_index("s")

    zero = jnp.zeros((16,), jnp.float32)

    def zrow(r, carry):
        for f in range(nf):
            rows[r, pl.ds(16 * f, 16)] = zero
        return carry

    def _idx_load(k, idxv, isem):
        return pltpu.async_copy(idx_hbm.at[k], idxv, isem)

    def _idx_wait(k, idxv, isem):
        pltpu.make_async_copy(idx_hbm.at[k], idxv, isem).wait()

    lax.fori_loop(0, CHUNK, zrow, 0)
    my_rows = pl.multiple_of(s * SUB_ROWS, SUB_ROWS)
    for j in range(SUB_ROWS // CHUNK):
        pltpu.sync_copy(rows.at[pl.ds(0, CHUNK)],
                        acc.at[pl.ds(my_rows + j * CHUNK, CHUNK)])
    plsc.subcore_barrier()

    n_pairs = base_chunks // 2

    def run(g_hbm, out_hbm):
        def process(idxv):
            pltpu.async_copy(g_hbm.at[idxv.at[0]], rows, gsem).wait()

            def scale_group(g, carry):
                emg = idxv[2, pl.ds(pl.multiple_of(g * 16, 16), 16)]
                for j in range(16):
                    scal = jnp.full(
                        (16,), lax.bitcast_convert_type(emg[j], jnp.float32))
                    e = g * 16 + j
                    for f in range(nf):
                        rows[e, pl.ds(16 * f, 16)] = rows[e, pl.ds(16 * f, 16)] * scal
                return carry

            lax.fori_loop(0, CHUNK // 16, scale_group, 0)
            pltpu.sync_copy(rows, acc.at[idxv.at[1]], add=True)

        _idx_load(s, idxv0, isem0)

        def loop(i, carry):
            ka = (2 * i) * N_SUB + s
            kb = (2 * i + 1) * N_SUB + s
            kc = (2 * i + 2) * N_SUB + s
            _idx_wait(ka, idxv0, isem0)
            _idx_load(kb, idxv1, isem1)
            process(idxv0)
            _idx_wait(kb, idxv1, isem1)

            @pl.when((i + 1 < n_pairs) | (s < rem_chunks))
            def _():
                _idx_load(kc, idxv0, isem0)

            process(idxv1)
            return carry

        lax.fori_loop(0, n_pairs, loop, 0)

        @pl.when(s < rem_chunks)
        def _():
            kt = base_chunks * N_SUB + s
            _idx_wait(kt, idxv0, isem0)
            process(idxv0)

        plsc.subcore_barrier()
        out_count_full = SUB_ROWS
        out_count_last = n_nodes - (N_SUB - 1) * SUB_ROWS

        @pl.when(s < N_SUB - 1)
        def _():
            pltpu.sync_copy(acc.at[pl.ds(my_rows, out_count_full)],
                            out_hbm.at[pl.ds(my_rows, out_count_full)])

        @pl.when(s == N_SUB - 1)
        def _():
            base_last = (N_SUB - 1) * SUB_ROWS
            pltpu.sync_copy(acc.at[pl.ds(base_last, out_count_last)],
                            out_hbm.at[pl.ds(base_last, out_count_last)])

    @pl.when(c == 0)
    def _():
        run(ga_hbm, outa_hbm)

    @pl.when(c == 1)
    def _():
        run(gb_hbm, outb_hbm)


@functools.lru_cache(maxsize=None)
def _build_segsum(n_nodes, n_edges, half):
    mesh = plsc.VectorSubcoreMesh(core_axis_name="c", subcore_axis_name="s")
    return pl.kernel(
        functools.partial(_segsum_body, (n_nodes, n_edges, half)),
        out_type=(jax.ShapeDtypeStruct((n_nodes, half), jnp.float32),
                  jax.ShapeDtypeStruct((n_nodes, half), jnp.float32)),
        mesh=mesh,
        scratch_types=[
            pltpu.VMEM_SHARED((ACC_ROWS, half), jnp.float32),
            pltpu.VMEM((3, CHUNK), jnp.int32),
            pltpu.VMEM((3, CHUNK), jnp.int32),
            pltpu.VMEM((CHUNK, half), jnp.float32),
            pltpu.SemaphoreType.DMA,
            pltpu.SemaphoreType.DMA,
            pltpu.SemaphoreType.DMA,
        ],
        name="segsum_sc",
    )


def _segsum_sc(ga, gb, idx3):
    return _build_segsum(ga.shape[0], idx3.shape[0] * CHUNK, ga.shape[1])(
        ga, gb, idx3)


def _pack_edges(src, dst, em_sig):
    n_chunks = src.shape[0] // CHUNK
    em_bits = lax.bitcast_convert_type(em_sig, jnp.int32)
    return jnp.stack([src.reshape(n_chunks, CHUNK), dst.reshape(n_chunks, CHUNK),
                      em_bits.reshape(n_chunks, CHUNK)], axis=1)


def _stage1_body(x_ref, nfm_ref, em_ref, w1a_ref, w1b_ref, ga_ref, gb_ref, ems_ref):
    sfm = jax.nn.sigmoid(nfm_ref[...])
    h = x_ref[...] * sfm
    ga_ref[...] = jnp.dot(h, w1a_ref[...], preferred_element_type=jnp.float32)
    gb_ref[...] = jnp.dot(h, w1b_ref[...], preferred_element_type=jnp.float32)
    ems_ref[...] = jax.nn.sigmoid(em_ref[...])


def _stage1(x, nfm2, em3, w1a, w1b):
    return pl.pallas_call(
        _stage1_body,
        grid=(N_ROW_BLKS,),
        in_specs=[
            pl.BlockSpec((ROW_BLK, D_FEAT), lambda i: (i, 0)),
            pl.BlockSpec((1, D_FEAT), lambda i: (0, 0)),
            pl.BlockSpec((1, 1, EM_BLK), lambda i: (i, 0, 0)),
            pl.BlockSpec((D_FEAT, HALF), lambda i: (0, 0)),
            pl.BlockSpec((D_FEAT, HALF), lambda i: (0, 0)),
        ],
        out_specs=[
            pl.BlockSpec((ROW_BLK, HALF), lambda i: (i, 0)),
            pl.BlockSpec((ROW_BLK, HALF), lambda i: (i, 0)),
            pl.BlockSpec((1, 1, EM_BLK), lambda i: (i, 0, 0)),
        ],
        out_shape=[
            jax.ShapeDtypeStruct((N_NODES, HALF), jnp.float32),
            jax.ShapeDtypeStruct((N_NODES, HALF), jnp.float32),
            jax.ShapeDtypeStruct((N_ROW_BLKS, 1, EM_BLK), jnp.float32),
        ],
    )(x, nfm2, em3, w1a, w1b)


def _stage2_body(aa_ref, ab_ref, w2_ref, ga_ref, gb_ref):
    h1a = jnp.maximum(aa_ref[...], 0.0)
    h1b = jnp.maximum(ab_ref[...], 0.0)
    w2 = w2_ref[...]
    ga_ref[...] = (jnp.dot(h1a, w2[:HALF, :HALF], preferred_element_type=jnp.float32)
                   + jnp.dot(h1b, w2[HALF:, :HALF], preferred_element_type=jnp.float32))
    gb_ref[...] = (jnp.dot(h1a, w2[:HALF, HALF:], preferred_element_type=jnp.float32)
                   + jnp.dot(h1b, w2[HALF:, HALF:], preferred_element_type=jnp.float32))


def _stage2(aa, ab, w2):
    return pl.pallas_call(
        _stage2_body,
        grid=(N_ROW_BLKS,),
        in_specs=[
            pl.BlockSpec((ROW_BLK, HALF), lambda i: (i, 0)),
            pl.BlockSpec((ROW_BLK, HALF), lambda i: (i, 0)),
            pl.BlockSpec((D_FEAT, D_FEAT), lambda i: (0, 0)),
        ],
        out_specs=[
            pl.BlockSpec((ROW_BLK, HALF), lambda i: (i, 0)),
            pl.BlockSpec((ROW_BLK, HALF), lambda i: (i, 0)),
        ],
        out_shape=[
            jax.ShapeDtypeStruct((N_NODES, HALF), jnp.float32),
            jax.ShapeDtypeStruct((N_NODES, HALF), jnp.float32),
        ],
    )(aa, ab, w2)


def _stage3_body(aa_ref, ab_ref, wout_ref, bi_ref, em_ref, pool_ref, stats_ref):
    i = pl.program_id(0)

    @pl.when(i == 0)
    def _():
        pool_ref[...] = jnp.zeros_like(pool_ref)
        stats_ref[...] = jnp.zeros_like(stats_ref)

    h2a = jnp.maximum(aa_ref[...], 0.0)
    h2b = jnp.maximum(ab_ref[...], 0.0)
    w = wout_ref[...]
    z = (jnp.dot(h2a, w[:HALF], preferred_element_type=jnp.float32)
         + jnp.dot(h2b, w[HALF:], preferred_element_type=jnp.float32))
    bi = bi_ref[0, 0, :]
    graphs = lax.broadcasted_iota(jnp.int32, (ROW_BLK, N_GRAPHS), 1)
    onehot = (bi[:, None] == graphs).astype(jnp.float32)
    zc = jnp.concatenate(
        [z, jnp.ones((ROW_BLK, 1), jnp.float32), jnp.zeros((ROW_BLK, 5), jnp.float32)],
        axis=1)
    pool_ref[...] += lax.dot_general(onehot, zc, (((0,), (0,)), ((), ())),
                                     preferred_element_type=jnp.float32)
    em = em_ref[...]
    s_em = jnp.sum(em)
    ent = -em * jnp.log(em + EPS) - (1.0 - em) * jnp.log(1.0 - em + EPS)
    s_ent = jnp.sum(ent)
    lane = lax.broadcasted_iota(jnp.int32, (1, 128), 1)
    stats_ref[...] += (jnp.where(lane == 0, s_em, 0.0)
                       + jnp.where(lane == 1, s_ent, 0.0))


def _stage3(aa, ab, wout, bi3, em3):
    return pl.pallas_call(
        _stage3_body,
        grid=(N_ROW_BLKS,),
        in_specs=[
            pl.BlockSpec((ROW_BLK, HALF), lambda i: (i, 0)),
            pl.BlockSpec((ROW_BLK, HALF), lambda i: (i, 0)),
            pl.BlockSpec((D_FEAT, N_CLASSES), lambda i: (0, 0)),
            pl.BlockSpec((1, 1, ROW_BLK), lambda i: (i, 0, 0)),
            pl.BlockSpec((1, 1, EM_BLK), lambda i: (i, 0, 0)),
        ],
        out_specs=[
            pl.BlockSpec((N_GRAPHS, 16), lambda i: (0, 0)),
            pl.BlockSpec((1, 128), lambda i: (0, 0)),
        ],
        out_shape=[
            jax.ShapeDtypeStruct((N_GRAPHS, 16), jnp.float32),
            jax.ShapeDtypeStruct((1, 128), jnp.float32),
        ],
    )(aa, ab, wout, bi3, em3)


def _stage4_body(pool_ref, stats_ref, nfm_ref, label_ref, out_ref):
    pool = pool_ref[...]
    counts = jnp.maximum(pool[:, 10:11], 1.0)
    logits = pool[:, :N_CLASSES] / counts
    mx = jnp.max(logits, axis=1, keepdims=True)
    ex = jnp.exp(logits - mx)
    lse = jnp.log(jnp.sum(ex, axis=1, keepdims=True)) + mx
    lbl = label_ref[0, 0]
    cls = lax.broadcasted_iota(jnp.int32, (N_GRAPHS, N_CLASSES), 1)
    sel = jnp.sum(jnp.where(cls == lbl, logits, 0.0), axis=1, keepdims=True)
    loss_pred = jnp.sum(lse - sel)
    s_em = stats_ref[0, 0]
    s_ent = stats_ref[0, 1]
    fm = jax.nn.sigmoid(nfm_ref[...])
    reg = 0.1 * s_em + s_ent / N_EDGES + jnp.mean(fm)
    loss = loss_pred + N_GRAPHS * reg
    out_ref[...] = jnp.broadcast_to(loss, (1, 1))


def _stage4(pool, stats, nfm2, label):
    return pl.pallas_call(
        _stage4_body,
        out_shape=jax.ShapeDtypeStruct((1, 1), jnp.float32),
    )(pool, stats, nfm2, label)


def kernel(x, edge_index, batch_index, expl_label, node_feat_mask, edge_mask,
           W1, W2, W_out):
    src = edge_index[0]
    dst = edge_index[1]
    nfm2 = node_feat_mask.reshape(1, D_FEAT)
    em3 = edge_mask.reshape(N_ROW_BLKS, 1, EM_BLK)
    ga, gb, ems3 = _stage1(x, nfm2, em3, W1[:, :HALF], W1[:, HALF:])
    em_sig = ems3.reshape(N_EDGES)
    idx3 = _pack_edges(src, dst, em_sig)
    aa, ab = _segsum_sc(ga, gb, idx3)
    g2a, g2b = _stage2(aa, ab, W2)
    a2a, a2b = _segsum_sc(g2a, g2b, idx3)
    bi3 = batch_index.reshape(N_ROW_BLKS, 1, ROW_BLK)
    pool, stats = _stage3(a2a, a2b, W_out, bi3, ems3)
    label = jnp.asarray(expl_label, jnp.int32).reshape(1, 1)
    out = _stage4(pool, stats, nfm2, label)
    return out.reshape(())


# speedup vs baseline: 2.2040x; 1.1938x over previous
import functools

import jax
import jax.numpy as jnp
from jax import lax
from jax.experimental import pallas as pl
from jax.experimental.pallas import tpu as pltpu
from jax.experimental.pallas import tpu_sc as plsc

N_NODES = 10000
N_EDGES = 160000
D_FEAT = 256
HALF = 128
N_CLASSES = 10
N_GRAPHS = 8
EPS = 1e-15

ROW_BLK = 200
N_ROW_BLKS = N_NODES // ROW_BLK
EM_BLK = N_EDGES // N_ROW_BLKS

CHUNK = 128
N_SUB = 16
SUB_ROWS = 640
ACC_ROWS = N_SUB * SUB_ROWS
IDX_BLK = 16
PAD_EDGES = N_SUB * IDX_BLK * CHUNK
DST_PAD = N_NODES


def _segsum_body(meta, ga_hbm, gb_hbm, idx_hbm,
                 outa_hbm, outb_hbm, acc, idxv0, idxv1, rows0, rows1,
                 gsem0, gsem1, isem0, isem1):
    n_nodes, n_edges, half = meta
    n_chunks = n_edges // CHUNK
    base_chunks = n_chunks // N_SUB
    rem_chunks = n_chunks % N_SUB
    nf = half // 16

    c = lax.axis_index("c")
    s = lax.axis_index("s")

    zero = jnp.zeros((16,), jnp.float32)

    def zrow(r, carry):
        for f in range(nf):
            rows0[r, pl.ds(16 * f, 16)] = zero
        return carry

    def _idx_load(k, idxv, isem):
        return pltpu.async_copy(idx_hbm.at[k], idxv, isem)

    def _idx_wait(k, idxv, isem):
        pltpu.make_async_copy(idx_hbm.at[k], idxv, isem).wait()

    lax.fori_loop(0, CHUNK, zrow, 0)
    my_rows = pl.multiple_of(s * SUB_ROWS, SUB_ROWS)
    for j in range(SUB_ROWS // CHUNK):
        pltpu.sync_copy(rows0.at[pl.ds(0, CHUNK)],
                        acc.at[pl.ds(my_rows + j * CHUNK, CHUNK)])
    plsc.subcore_barrier()

    n_pairs = base_chunks // 2

    def run(g_hbm, out_hbm):
        def _gather(idxv, rows, gsem):
            return pltpu.async_copy(g_hbm.at[idxv.at[0]], rows, gsem)

        def _gather_wait(idxv, rows, gsem):
            pltpu.make_async_copy(g_hbm.at[idxv.at[0]], rows, gsem).wait()

        def scale_scatter(idxv, rows):
            def scale_group(g, carry):
                emg = idxv[2, pl.ds(pl.multiple_of(g * 16, 16), 16)]
                for j in range(16):
                    scal = jnp.full(
                        (16,), lax.bitcast_convert_type(emg[j], jnp.float32))
                    e = g * 16 + j
                    for f in range(nf):
                        rows[e, pl.ds(16 * f, 16)] = rows[e, pl.ds(16 * f, 16)] * scal
                return carry

            lax.fori_loop(0, CHUNK // 16, scale_group, 0)
            pltpu.sync_copy(rows, acc.at[idxv.at[1]], add=True)

        _idx_load(s, idxv0, isem0)
        _idx_wait(s, idxv0, isem0)
        _gather(idxv0, rows0, gsem0)
        _idx_load(N_SUB + s, idxv1, isem1)

        def loop(i, carry):
            ka = (2 * i) * N_SUB + s
            kb = (2 * i + 1) * N_SUB + s
            kc = (2 * i + 2) * N_SUB + s
            kd = (2 * i + 3) * N_SUB + s
            have_kc = (i + 1 < n_pairs) | (s < rem_chunks)
            _idx_wait(kb, idxv1, isem1)
            _gather(idxv1, rows1, gsem1)
            _gather_wait(idxv0, rows0, gsem0)
            scale_scatter(idxv0, rows0)

            @pl.when(have_kc)
            def _():
                _idx_load(kc, idxv0, isem0)

            _gather_wait(idxv1, rows1, gsem1)
            scale_scatter(idxv1, rows1)

            @pl.when(have_kc)
            def _():
                _idx_wait(kc, idxv0, isem0)
                _gather(idxv0, rows0, gsem0)

            @pl.when(i + 1 < n_pairs)
            def _():
                _idx_load(kd, idxv1, isem1)

            return carry

        lax.fori_loop(0, n_pairs, loop, 0)

        @pl.when(s < rem_chunks)
        def _():
            _gather_wait(idxv0, rows0, gsem0)
            scale_scatter(idxv0, rows0)

        plsc.subcore_barrier()
        out_count_full = SUB_ROWS
        out_count_last = n_nodes - (N_SUB - 1) * SUB_ROWS

        @pl.when(s < N_SUB - 1)
        def _():
            pltpu.sync_copy(acc.at[pl.ds(my_rows, out_count_full)],
                            out_hbm.at[pl.ds(my_rows, out_count_full)])

        @pl.when(s == N_SUB - 1)
        def _():
            base_last = (N_SUB - 1) * SUB_ROWS
            pltpu.sync_copy(acc.at[pl.ds(base_last, out_count_last)],
                            out_hbm.at[pl.ds(base_last, out_count_last)])

    @pl.when(c == 0)
    def _():
        run(ga_hbm, outa_hbm)

    @pl.when(c == 1)
    def _():
        run(gb_hbm, outb_hbm)


@functools.lru_cache(maxsize=None)
def _build_segsum(n_nodes, n_edges, half):
    mesh = plsc.VectorSubcoreMesh(core_axis_name="c", subcore_axis_name="s")
    return pl.kernel(
        functools.partial(_segsum_body, (n_nodes, n_edges, half)),
        out_type=(jax.ShapeDtypeStruct((n_nodes, half), jnp.float32),
                  jax.ShapeDtypeStruct((n_nodes, half), jnp.float32)),
        mesh=mesh,
        scratch_types=[
            pltpu.VMEM_SHARED((ACC_ROWS, half), jnp.float32),
            pltpu.VMEM((3, CHUNK), jnp.int32),
            pltpu.VMEM((3, CHUNK), jnp.int32),
            pltpu.VMEM((CHUNK, half), jnp.float32),
            pltpu.VMEM((CHUNK, half), jnp.float32),
            pltpu.SemaphoreType.DMA,
            pltpu.SemaphoreType.DMA,
            pltpu.SemaphoreType.DMA,
            pltpu.SemaphoreType.DMA,
        ],
        name="segsum_sc",
    )


def _segsum_sc(ga, gb, idx3):
    return _build_segsum(ga.shape[0], idx3.shape[0] * CHUNK, ga.shape[1])(
        ga, gb, idx3)


def _pack_edges(src, dst, em_sig):
    n_chunks = src.shape[0] // CHUNK
    em_bits = lax.bitcast_convert_type(em_sig, jnp.int32)
    return jnp.stack([src.reshape(n_chunks, CHUNK), dst.reshape(n_chunks, CHUNK),
                      em_bits.reshape(n_chunks, CHUNK)], axis=1)


def _stage1_body(x_ref, nfm_ref, em_ref, w1a_ref, w1b_ref, ga_ref, gb_ref, ems_ref):
    sfm = jax.nn.sigmoid(nfm_ref[...])
    h = x_ref[...] * sfm
    ga_ref[...] = jnp.dot(h, w1a_ref[...], preferred_element_type=jnp.float32)
    gb_ref[...] = jnp.dot(h, w1b_ref[...], preferred_element_type=jnp.float32)
    ems_ref[...] = jax.nn.sigmoid(em_ref[...])


def _stage1(x, nfm2, em3, w1a, w1b):
    return pl.pallas_call(
        _stage1_body,
        grid=(N_ROW_BLKS,),
        in_specs=[
            pl.BlockSpec((ROW_BLK, D_FEAT), lambda i: (i, 0)),
            pl.BlockSpec((1, D_FEAT), lambda i: (0, 0)),
            pl.BlockSpec((1, 1, EM_BLK), lambda i: (i, 0, 0)),
            pl.BlockSpec((D_FEAT, HALF), lambda i: (0, 0)),
            pl.BlockSpec((D_FEAT, HALF), lambda i: (0, 0)),
        ],
        out_specs=[
            pl.BlockSpec((ROW_BLK, HALF), lambda i: (i, 0)),
            pl.BlockSpec((ROW_BLK, HALF), lambda i: (i, 0)),
            pl.BlockSpec((1, 1, EM_BLK), lambda i: (i, 0, 0)),
        ],
        out_shape=[
            jax.ShapeDtypeStruct((N_NODES, HALF), jnp.float32),
            jax.ShapeDtypeStruct((N_NODES, HALF), jnp.float32),
            jax.ShapeDtypeStruct((N_ROW_BLKS, 1, EM_BLK), jnp.float32),
        ],
    )(x, nfm2, em3, w1a, w1b)


def _stage2_body(aa_ref, ab_ref, w2_ref, ga_ref, gb_ref):
    h1a = jnp.maximum(aa_ref[...], 0.0)
    h1b = jnp.maximum(ab_ref[...], 0.0)
    w2 = w2_ref[...]
    ga_ref[...] = (jnp.dot(h1a, w2[:HALF, :HALF], preferred_element_type=jnp.float32)
                   + jnp.dot(h1b, w2[HALF:, :HALF], preferred_element_type=jnp.float32))
    gb_ref[...] = (jnp.dot(h1a, w2[:HALF, HALF:], preferred_element_type=jnp.float32)
                   + jnp.dot(h1b, w2[HALF:, HALF:], preferred_element_type=jnp.float32))


def _stage2(aa, ab, w2):
    return pl.pallas_call(
        _stage2_body,
        grid=(N_ROW_BLKS,),
        in_specs=[
            pl.BlockSpec((ROW_BLK, HALF), lambda i: (i, 0)),
            pl.BlockSpec((ROW_BLK, HALF), lambda i: (i, 0)),
            pl.BlockSpec((D_FEAT, D_FEAT), lambda i: (0, 0)),
        ],
        out_specs=[
            pl.BlockSpec((ROW_BLK, HALF), lambda i: (i, 0)),
            pl.BlockSpec((ROW_BLK, HALF), lambda i: (i, 0)),
        ],
        out_shape=[
            jax.ShapeDtypeStruct((N_NODES, HALF), jnp.float32),
            jax.ShapeDtypeStruct((N_NODES, HALF), jnp.float32),
        ],
    )(aa, ab, w2)


def _stage3_body(aa_ref, ab_ref, wout_ref, bi_ref, em_ref, pool_ref, stats_ref):
    i = pl.program_id(0)

    @pl.when(i == 0)
    def _():
        pool_ref[...] = jnp.zeros_like(pool_ref)
        stats_ref[...] = jnp.zeros_like(stats_ref)

    h2a = jnp.maximum(aa_ref[...], 0.0)
    h2b = jnp.maximum(ab_ref[...], 0.0)
    w = wout_ref[...]
    z = (jnp.dot(h2a, w[:HALF], preferred_element_type=jnp.float32)
         + jnp.dot(h2b, w[HALF:], preferred_element_type=jnp.float32))
    bi = bi_ref[0, 0, :]
    graphs = lax.broadcasted_iota(jnp.int32, (ROW_BLK, N_GRAPHS), 1)
    onehot = (bi[:, None] == graphs).astype(jnp.float32)
    zc = jnp.concatenate(
        [z, jnp.ones((ROW_BLK, 1), jnp.float32), jnp.zeros((ROW_BLK, 5), jnp.float32)],
        axis=1)
    pool_ref[...] += lax.dot_general(onehot, zc, (((0,), (0,)), ((), ())),
                                     preferred_element_type=jnp.float32)
    em = em_ref[...]
    s_em = jnp.sum(em)
    ent = -em * jnp.log(em + EPS) - (1.0 - em) * jnp.log(1.0 - em + EPS)
    s_ent = jnp.sum(ent)
    lane = lax.broadcasted_iota(jnp.int32, (1, 128), 1)
    stats_ref[...] += (jnp.where(lane == 0, s_em, 0.0)
                       + jnp.where(lane == 1, s_ent, 0.0))


def _stage3(aa, ab, wout, bi3, em3):
    return pl.pallas_call(
        _stage3_body,
        grid=(N_ROW_BLKS,),
        in_specs=[
            pl.BlockSpec((ROW_BLK, HALF), lambda i: (i, 0)),
            pl.BlockSpec((ROW_BLK, HALF), lambda i: (i, 0)),
            pl.BlockSpec((D_FEAT, N_CLASSES), lambda i: (0, 0)),
            pl.BlockSpec((1, 1, ROW_BLK), lambda i: (i, 0, 0)),
            pl.BlockSpec((1, 1, EM_BLK), lambda i: (i, 0, 0)),
        ],
        out_specs=[
            pl.BlockSpec((N_GRAPHS, 16), lambda i: (0, 0)),
            pl.BlockSpec((1, 128), lambda i: (0, 0)),
        ],
        out_shape=[
            jax.ShapeDtypeStruct((N_GRAPHS, 16), jnp.float32),
            jax.ShapeDtypeStruct((1, 128), jnp.float32),
        ],
    )(aa, ab, wout, bi3, em3)


def _stage4_body(pool_ref, stats_ref, nfm_ref, label_ref, out_ref):
    pool = pool_ref[...]
    counts = jnp.maximum(pool[:, 10:11], 1.0)
    logits = pool[:, :N_CLASSES] / counts
    mx = jnp.max(logits, axis=1, keepdims=True)
    ex = jnp.exp(logits - mx)
    lse = jnp.log(jnp.sum(ex, axis=1, keepdims=True)) + mx
    lbl = label_ref[0, 0]
    cls = lax.broadcasted_iota(jnp.int32, (N_GRAPHS, N_CLASSES), 1)
    sel = jnp.sum(jnp.where(cls == lbl, logits, 0.0), axis=1, keepdims=True)
    loss_pred = jnp.sum(lse - sel)
    s_em = stats_ref[0, 0]
    s_ent = stats_ref[0, 1]
    fm = jax.nn.sigmoid(nfm_ref[...])
    reg = 0.1 * s_em + s_ent / N_EDGES + jnp.mean(fm)
    loss = loss_pred + N_GRAPHS * reg
    out_ref[...] = jnp.broadcast_to(loss, (1, 1))


def _stage4(pool, stats, nfm2, label):
    return pl.pallas_call(
        _stage4_body,
        out_shape=jax.ShapeDtypeStruct((1, 1), jnp.float32),
    )(pool, stats, nfm2, label)


def kernel(x, edge_index, batch_index, expl_label, node_feat_mask, edge_mask,
           W1, W2, W_out):
    src = edge_index[0]
    dst = edge_index[1]
    nfm2 = node_feat_mask.reshape(1, D_FEAT)
    em3 = edge_mask.reshape(N_ROW_BLKS, 1, EM_BLK)
    ga, gb, ems3 = _stage1(x, nfm2, em3, W1[:, :HALF], W1[:, HALF:])
    em_sig = ems3.reshape(N_EDGES)
    idx3 = _pack_edges(src, dst, em_sig)
    aa, ab = _segsum_sc(ga, gb, idx3)
    g2a, g2b = _stage2(aa, ab, W2)
    a2a, a2b = _segsum_sc(g2a, g2b, idx3)
    bi3 = batch_index.reshape(N_ROW_BLKS, 1, ROW_BLK)
    pool, stats = _stage3(a2a, a2b, W_out, bi3, ems3)
    label = jnp.asarray(expl_label, jnp.int32).reshape(1, 1)
    out = _stage4(pool, stats, nfm2, label)
    return out.reshape(())


# speedup vs baseline: 2.2050x; 1.0005x over previous
import functools

import jax
import jax.numpy as jnp
from jax import lax
from jax.experimental import pallas as pl
from jax.experimental.pallas import tpu as pltpu
from jax.experimental.pallas import tpu_sc as plsc

N_NODES = 10000
N_EDGES = 160000
D_FEAT = 256
HALF = 128
N_CLASSES = 10
N_GRAPHS = 8
EPS = 1e-15

ROW_BLK = 200
N_ROW_BLKS = N_NODES // ROW_BLK
EM_BLK = N_EDGES // N_ROW_BLKS

CHUNK = 128
N_SUB = 16
SUB_ROWS = 640
ACC_ROWS = N_SUB * SUB_ROWS


def _segsum_body(meta, ga_hbm, gb_hbm, idx_hbm,
                 outa_hbm, outb_hbm, acc, idxv0, idxv1, rows0, rows1,
                 gsem0, gsem1, isem0, isem1):
    n_nodes, n_edges, half = meta
    n_chunks = n_edges // CHUNK
    base_chunks = n_chunks // N_SUB
    rem_chunks = n_chunks % N_SUB
    nf = half // 16

    c = lax.axis_index("c")
    s = lax.axis_index("s")

    zero = jnp.zeros((16,), jnp.float32)

    def zrow(r, carry):
        for f in range(nf):
            rows0[r, pl.ds(16 * f, 16)] = zero
        return carry

    def _idx_load(k, idxv, isem):
        return pltpu.async_copy(idx_hbm.at[k], idxv, isem)

    def _idx_wait(k, idxv, isem):
        pltpu.make_async_copy(idx_hbm.at[k], idxv, isem).wait()

    lax.fori_loop(0, CHUNK, zrow, 0)
    my_rows = pl.multiple_of(s * SUB_ROWS, SUB_ROWS)
    for j in range(SUB_ROWS // CHUNK):
        pltpu.sync_copy(rows0.at[pl.ds(0, CHUNK)],
                        acc.at[pl.ds(my_rows + j * CHUNK, CHUNK)])
    plsc.subcore_barrier()

    n_pairs = base_chunks // 2

    def run(g_hbm, out_hbm):
        def _gather(idxv, rows, gsem):
            return pltpu.async_copy(g_hbm.at[idxv.at[0]], rows, gsem)

        def _gather_wait(idxv, rows, gsem):
            pltpu.make_async_copy(g_hbm.at[idxv.at[0]], rows, gsem).wait()

        def scale_scatter(idxv, rows):
            def scale_group(g, carry):
                emg = idxv[2, pl.ds(pl.multiple_of(g * 16, 16), 16)]
                for j in range(16):
                    scal = jnp.full(
                        (16,), lax.bitcast_convert_type(emg[j], jnp.float32))
                    e = g * 16 + j
                    for f in range(nf):
                        rows[e, pl.ds(16 * f, 16)] = rows[e, pl.ds(16 * f, 16)] * scal
                return carry

            lax.fori_loop(0, CHUNK // 16, scale_group, 0)
            pltpu.sync_copy(rows, acc.at[idxv.at[1]], add=True)

        _idx_load(s, idxv0, isem0)
        _idx_wait(s, idxv0, isem0)
        _gather(idxv0, rows0, gsem0)
        _idx_load(N_SUB + s, idxv1, isem1)

        def loop(i, carry):
            ka = (2 * i) * N_SUB + s
            kb = (2 * i + 1) * N_SUB + s
            kc = (2 * i + 2) * N_SUB + s
            kd = (2 * i + 3) * N_SUB + s
            have_kc = (i + 1 < n_pairs) | (s < rem_chunks)
            _idx_wait(kb, idxv1, isem1)
            _gather(idxv1, rows1, gsem1)
            _gather_wait(idxv0, rows0, gsem0)
            scale_scatter(idxv0, rows0)

            @pl.when(have_kc)
            def _():
                _idx_load(kc, idxv0, isem0)

            _gather_wait(idxv1, rows1, gsem1)
            scale_scatter(idxv1, rows1)

            @pl.when(have_kc)
            def _():
                _idx_wait(kc, idxv0, isem0)
                _gather(idxv0, rows0, gsem0)

            @pl.when(i + 1 < n_pairs)
            def _():
                _idx_load(kd, idxv1, isem1)

            return carry

        lax.fori_loop(0, n_pairs, loop, 0)

        @pl.when(s < rem_chunks)
        def _():
            _gather_wait(idxv0, rows0, gsem0)
            scale_scatter(idxv0, rows0)

        plsc.subcore_barrier()
        out_count_full = SUB_ROWS
        out_count_last = n_nodes - (N_SUB - 1) * SUB_ROWS

        @pl.when(s < N_SUB - 1)
        def _():
            pltpu.sync_copy(acc.at[pl.ds(my_rows, out_count_full)],
                            out_hbm.at[pl.ds(my_rows, out_count_full)])

        @pl.when(s == N_SUB - 1)
        def _():
            base_last = (N_SUB - 1) * SUB_ROWS
            pltpu.sync_copy(acc.at[pl.ds(base_last, out_count_last)],
                            out_hbm.at[pl.ds(base_last, out_count_last)])

    @pl.when(c == 0)
    def _():
        run(ga_hbm, outa_hbm)

    @pl.when(c == 1)
    def _():
        run(gb_hbm, outb_hbm)


@functools.lru_cache(maxsize=None)
def _build_segsum(n_nodes, n_edges, half):
    mesh = plsc.VectorSubcoreMesh(core_axis_name="c", subcore_axis_name="s")
    return pl.kernel(
        functools.partial(_segsum_body, (n_nodes, n_edges, half)),
        out_type=(jax.ShapeDtypeStruct((n_nodes, half), jnp.float32),
                  jax.ShapeDtypeStruct((n_nodes, half), jnp.float32)),
        mesh=mesh,
        scratch_types=[
            pltpu.VMEM_SHARED((ACC_ROWS, half), jnp.float32),
            pltpu.VMEM((3, CHUNK), jnp.int32),
            pltpu.VMEM((3, CHUNK), jnp.int32),
            pltpu.VMEM((CHUNK, half), jnp.float32),
            pltpu.VMEM((CHUNK, half), jnp.float32),
            pltpu.SemaphoreType.DMA,
            pltpu.SemaphoreType.DMA,
            pltpu.SemaphoreType.DMA,
            pltpu.SemaphoreType.DMA,
        ],
        name="segsum_sc",
    )


def _segsum_sc(ga, gb, idx3):
    return _build_segsum(ga.shape[0], idx3.shape[0] * CHUNK, ga.shape[1])(
        ga, gb, idx3)


def _pack_edges(src, dst, em_sig):
    n_chunks = src.shape[0] // CHUNK
    em_bits = lax.bitcast_convert_type(em_sig, jnp.int32)
    return jnp.stack([src.reshape(n_chunks, CHUNK), dst.reshape(n_chunks, CHUNK),
                      em_bits.reshape(n_chunks, CHUNK)], axis=1)


def _stage1_body(x_ref, nfm_ref, em_ref, w1a_ref, w1b_ref, ga_ref, gb_ref, ems_ref):
    sfm = jax.nn.sigmoid(nfm_ref[...])
    h = x_ref[...] * sfm
    ga_ref[...] = jnp.dot(h, w1a_ref[...], preferred_element_type=jnp.float32)
    gb_ref[...] = jnp.dot(h, w1b_ref[...], preferred_element_type=jnp.float32)
    ems_ref[...] = jax.nn.sigmoid(em_ref[...])


def _stage1(x, nfm2, em3, w1a, w1b):
    return pl.pallas_call(
        _stage1_body,
        grid=(N_ROW_BLKS,),
        in_specs=[
            pl.BlockSpec((ROW_BLK, D_FEAT), lambda i: (i, 0)),
            pl.BlockSpec((1, D_FEAT), lambda i: (0, 0)),
            pl.BlockSpec((1, 1, EM_BLK), lambda i: (i, 0, 0)),
            pl.BlockSpec((D_FEAT, HALF), lambda i: (0, 0)),
            pl.BlockSpec((D_FEAT, HALF), lambda i: (0, 0)),
        ],
        out_specs=[
            pl.BlockSpec((ROW_BLK, HALF), lambda i: (i, 0)),
            pl.BlockSpec((ROW_BLK, HALF), lambda i: (i, 0)),
            pl.BlockSpec((1, 1, EM_BLK), lambda i: (i, 0, 0)),
        ],
        out_shape=[
            jax.ShapeDtypeStruct((N_NODES, HALF), jnp.float32),
            jax.ShapeDtypeStruct((N_NODES, HALF), jnp.float32),
            jax.ShapeDtypeStruct((N_ROW_BLKS, 1, EM_BLK), jnp.float32),
        ],
    )(x, nfm2, em3, w1a, w1b)


def _stage2_body(aa_ref, ab_ref, w2_ref, ga_ref, gb_ref):
    h1a = jnp.maximum(aa_ref[...], 0.0)
    h1b = jnp.maximum(ab_ref[...], 0.0)
    w2 = w2_ref[...]
    ga_ref[...] = (jnp.dot(h1a, w2[:HALF, :HALF], preferred_element_type=jnp.float32)
                   + jnp.dot(h1b, w2[HALF:, :HALF], preferred_element_type=jnp.float32))
    gb_ref[...] = (jnp.dot(h1a, w2[:HALF, HALF:], preferred_element_type=jnp.float32)
                   + jnp.dot(h1b, w2[HALF:, HALF:], preferred_element_type=jnp.float32))


def _stage2(aa, ab, w2):
    return pl.pallas_call(
        _stage2_body,
        grid=(N_ROW_BLKS,),
        in_specs=[
            pl.BlockSpec((ROW_BLK, HALF), lambda i: (i, 0)),
            pl.BlockSpec((ROW_BLK, HALF), lambda i: (i, 0)),
            pl.BlockSpec((D_FEAT, D_FEAT), lambda i: (0, 0)),
        ],
        out_specs=[
            pl.BlockSpec((ROW_BLK, HALF), lambda i: (i, 0)),
            pl.BlockSpec((ROW_BLK, HALF), lambda i: (i, 0)),
        ],
        out_shape=[
            jax.ShapeDtypeStruct((N_NODES, HALF), jnp.float32),
            jax.ShapeDtypeStruct((N_NODES, HALF), jnp.float32),
        ],
    )(aa, ab, w2)


def _stage3_body(aa_ref, ab_ref, wout_ref, bi_ref, em_ref, pool_ref, stats_ref):
    i = pl.program_id(0)

    @pl.when(i == 0)
    def _():
        pool_ref[...] = jnp.zeros_like(pool_ref)
        stats_ref[...] = jnp.zeros_like(stats_ref)

    h2a = jnp.maximum(aa_ref[...], 0.0)
    h2b = jnp.maximum(ab_ref[...], 0.0)
    w = wout_ref[...]
    z = (jnp.dot(h2a, w[:HALF], preferred_element_type=jnp.float32)
         + jnp.dot(h2b, w[HALF:], preferred_element_type=jnp.float32))
    bi = bi_ref[0, 0, :]
    graphs = lax.broadcasted_iota(jnp.int32, (ROW_BLK, N_GRAPHS), 1)
    onehot = (bi[:, None] == graphs).astype(jnp.float32)
    zc = jnp.concatenate(
        [z, jnp.ones((ROW_BLK, 1), jnp.float32), jnp.zeros((ROW_BLK, 5), jnp.float32)],
        axis=1)
    pool_ref[...] += lax.dot_general(onehot, zc, (((0,), (0,)), ((), ())),
                                     preferred_element_type=jnp.float32)
    em = em_ref[...]
    s_em = jnp.sum(em)
    ent = -em * jnp.log(em + EPS) - (1.0 - em) * jnp.log(1.0 - em + EPS)
    s_ent = jnp.sum(ent)
    lane = lax.broadcasted_iota(jnp.int32, (1, 128), 1)
    stats_ref[...] += (jnp.where(lane == 0, s_em, 0.0)
                       + jnp.where(lane == 1, s_ent, 0.0))


def _stage3(aa, ab, wout, bi3, em3):
    return pl.pallas_call(
        _stage3_body,
        grid=(N_ROW_BLKS,),
        in_specs=[
            pl.BlockSpec((ROW_BLK, HALF), lambda i: (i, 0)),
            pl.BlockSpec((ROW_BLK, HALF), lambda i: (i, 0)),
            pl.BlockSpec((D_FEAT, N_CLASSES), lambda i: (0, 0)),
            pl.BlockSpec((1, 1, ROW_BLK), lambda i: (i, 0, 0)),
            pl.BlockSpec((1, 1, EM_BLK), lambda i: (i, 0, 0)),
        ],
        out_specs=[
            pl.BlockSpec((N_GRAPHS, 16), lambda i: (0, 0)),
            pl.BlockSpec((1, 128), lambda i: (0, 0)),
        ],
        out_shape=[
            jax.ShapeDtypeStruct((N_GRAPHS, 16), jnp.float32),
            jax.ShapeDtypeStruct((1, 128), jnp.float32),
        ],
    )(aa, ab, wout, bi3, em3)


def _stage4_body(pool_ref, stats_ref, nfm_ref, label_ref, out_ref):
    pool = pool_ref[...]
    counts = jnp.maximum(pool[:, 10:11], 1.0)
    logits = pool[:, :N_CLASSES] / counts
    mx = jnp.max(logits, axis=1, keepdims=True)
    ex = jnp.exp(logits - mx)
    lse = jnp.log(jnp.sum(ex, axis=1, keepdims=True)) + mx
    lbl = label_ref[0, 0]
    cls = lax.broadcasted_iota(jnp.int32, (N_GRAPHS, N_CLASSES), 1)
    sel = jnp.sum(jnp.where(cls == lbl, logits, 0.0), axis=1, keepdims=True)
    loss_pred = jnp.sum(lse - sel)
    s_em = stats_ref[0, 0]
    s_ent = stats_ref[0, 1]
    fm = jax.nn.sigmoid(nfm_ref[...])
    reg = 0.1 * s_em + s_ent / N_EDGES + jnp.mean(fm)
    loss = loss_pred + N_GRAPHS * reg
    out_ref[...] = jnp.broadcast_to(loss, (1, 1))


def _stage4(pool, stats, nfm2, label):
    return pl.pallas_call(
        _stage4_body,
        out_shape=jax.ShapeDtypeStruct((1, 1), jnp.float32),
    )(pool, stats, nfm2, label)


def kernel(x, edge_index, batch_index, expl_label, node_feat_mask, edge_mask,
           W1, W2, W_out):
    src = edge_index[0]
    dst = edge_index[1]
    nfm2 = node_feat_mask.reshape(1, D_FEAT)
    em3 = edge_mask.reshape(N_ROW_BLKS, 1, EM_BLK)
    ga, gb, ems3 = _stage1(x, nfm2, em3, W1[:, :HALF], W1[:, HALF:])
    em_sig = ems3.reshape(N_EDGES)
    idx3 = _pack_edges(src, dst, em_sig)
    aa, ab = _segsum_sc(ga, gb, idx3)
    g2a, g2b = _stage2(aa, ab, W2)
    a2a, a2b = _segsum_sc(g2a, g2b, idx3)
    bi3 = batch_index.reshape(N_ROW_BLKS, 1, ROW_BLK)
    pool, stats = _stage3(a2a, a2b, W_out, bi3, ems3)
    label = jnp.asarray(expl_label, jnp.int32).reshape(1, 1)
    out = _stage4(pool, stats, nfm2, label)
    return out.reshape(())
